# baseline (speedup 1.0000x reference)
import functools

import jax
import jax.numpy as jnp
from jax import lax
from jax.experimental import pallas as pl
from jax.experimental.pallas import tpu as pltpu, tpu_sc as plsc

B, Q, C, D, T = 4, 900, 91, 256, 25
P = 2 * B
TPAD = 32
_HIGH = jax.lax.Precision.HIGHEST
_INTERPRET = False


def _cost_T(logits, bT, lab_col, tbox):
    m = jnp.max(logits, axis=1, keepdims=True)
    e = jnp.exp(logits - m)
    prob = e / jnp.sum(e, axis=1, keepdims=True)
    cls_iota = jax.lax.broadcasted_iota(jnp.int32, (T, C), 1)
    onehot = (lab_col == cls_iota).astype(jnp.float32)
    g = jax.lax.dot_general(onehot, prob, (((1,), (1,)), ((), ())),
                            precision=_HIGH)
    cost = -2.0 * g
    for k in range(4):
        cost = cost + 5.0 * jnp.abs(tbox[:, k:k + 1] - bT[k:k + 1, :])
    return cost


def _match_body(lg_p, bT_p, lg_s, bT_s, lab, tb, out_ref):
    costs = []
    for lg, bT in ((lg_p, bT_p), (lg_s, bT_s)):
        for b in range(B):
            costs.append(_cost_T(lg[b], bT[b], lab[b], tb[b]))
    cost3 = jnp.stack(costs, axis=1)

    iota_q = jax.lax.broadcasted_iota(jnp.int32, (P, Q), 1)
    tcol = jax.lax.broadcasted_iota(jnp.int32, (P, T), 1)
    avail = jnp.ones((P, Q), jnp.float32)
    I = jnp.zeros((P, T), jnp.int32)
    for t in range(T):
        col = jnp.where(avail > 0.0, cost3[t], jnp.inf)
        mval = jnp.min(col, axis=1, keepdims=True)
        idx = jnp.min(jnp.where(col == mval, iota_q, jnp.int32(2 ** 30)),
                      axis=1, keepdims=True)
        avail = jnp.where(iota_q == idx, 0.0, avail)
        I = jnp.where(tcol == t, idx, I)

    base = (jax.lax.broadcasted_iota(jnp.int32, (P, T), 0) % B) * Q
    out_ref[:, 0:T] = I + base
    out_ref[:, T:TPAD] = jnp.zeros((P, TPAD - T), jnp.int32)


def _newton_rsqrt(x):
    xi = plsc.bitcast(x, jnp.int32)
    y = plsc.bitcast(jnp.int32(0x5F3759DF) - (xi >> 1), jnp.float32)
    for _ in range(4):
        y = y * (1.5 - 0.5 * x * y * y)
    return y


_SEG = 7


def _sc_loss_body(idx_hbm, qp_hbm, qs_hbm, out_hbm,
                  idx_v, rows_v, f1_v, f2_v, red_v, red16_v,
                  feats_sh, csum_sh, sem):
    c = lax.axis_index("c")
    s = lax.axis_index("s")
    lane = lax.iota(jnp.int32, 16)

    @pl.when((c == 0) & (s < P))
    def _gather():
        pltpu.sync_copy(idx_hbm.at[s], idx_v)

        @pl.when(s < B)
        def _():
            pltpu.async_copy(qp_hbm.at[idx_v], rows_v, sem).wait()

        @pl.when(s >= B)
        def _():
            pltpu.async_copy(qs_hbm.at[idx_v], rows_v, sem).wait()

        pltpu.sync_copy(rows_v, feats_sh.at[s])

    plsc.subcore_barrier()

    @pl.when(c == 0)
    def _pair_and_reduce():
        b = s // 4
        tlo = (s % 4) * _SEG
        pltpu.sync_copy(feats_sh.at[b, pl.ds(tlo, _SEG)], f1_v)
        pltpu.sync_copy(feats_sh.at[b + B, pl.ds(tlo, _SEG)], f2_v)
        adv = jnp.zeros((16,), jnp.float32)
        a1v = jnp.ones((16,), jnp.float32)
        a2v = jnp.ones((16,), jnp.float32)
        for tl in range(_SEG):
            ad = jnp.zeros((16,), jnp.float32)
            a1 = jnp.zeros((16,), jnp.float32)
            a2 = jnp.zeros((16,), jnp.float32)
            for dc in range(D // 16):
                f1 = f1_v[tl, pl.ds(dc * 16, 16)]
                f2 = f2_v[tl, pl.ds(dc * 16, 16)]
                ad = ad + f1 * f2
                a1 = a1 + f1 * f1
                a2 = a2 + f2 * f2
            sel = lane == tl
            adv = jnp.where(sel, jnp.sum(ad), adv)
            a1v = jnp.where(sel, jnp.sum(a1), a1v)
            a2v = jnp.where(sel, jnp.sum(a2), a2v)
        cosv = adv * _newton_rsqrt(a1v * a2v)
        valid = (lane < _SEG) & ((tlo + lane) < T)
        csc = jnp.sum(jnp.where(valid, cosv, 0.0))
        red_v[...] = jnp.where(lane == 0, csc, 0.0)
        pltpu.sync_copy(red_v, csum_sh.at[s])

    plsc.subcore_barrier()

    @pl.when((c == 0) & (s == 0))
    def _finalize():
        pltpu.sync_copy(csum_sh, red16_v)
        tot = jnp.zeros((16,), jnp.float32)
        for w in range(16):
            tot = tot + red16_v[w]
        total = jnp.sum(tot)
        red_v[...] = jnp.where(lane == 0, -total * (1.0 / (B * T)), 0.0)
        pltpu.sync_copy(red_v, out_hbm)


def _make_sc_loss():
    return functools.partial(
        pl.kernel,
        out_type=jax.ShapeDtypeStruct((16,), jnp.float32),
        mesh=plsc.VectorSubcoreMesh(core_axis_name="c", subcore_axis_name="s"),
        compiler_params=pltpu.CompilerParams(use_tc_tiling_on_sc=False,
                                             needs_layout_passes=False),
        interpret=_INTERPRET,
        scratch_types=[
            pltpu.VMEM((TPAD,), jnp.int32),
            pltpu.VMEM((TPAD, D), jnp.float32),
            pltpu.VMEM((_SEG, D), jnp.float32),
            pltpu.VMEM((_SEG, D), jnp.float32),
            pltpu.VMEM((16,), jnp.float32),
            pltpu.VMEM((16, 16), jnp.float32),
            pltpu.VMEM_SHARED((P, TPAD, D), jnp.float32),
            pltpu.VMEM_SHARED((16, 16), jnp.float32),
            pltpu.SemaphoreType.DMA,
        ],
    )(_sc_loss_body)


@jax.jit
def kernel(pred_logits, pred_boxes, pred_queries, siamese_logits,
           siamese_boxes, siamese_query, tgt_labels, tgt_boxes):
    bT_p = pred_boxes.transpose(0, 2, 1)
    bT_s = siamese_boxes.transpose(0, 2, 1)
    lab = tgt_labels.astype(jnp.int32).reshape(B, T, 1)
    idx = pl.pallas_call(
        _match_body,
        out_shape=jax.ShapeDtypeStruct((P, TPAD), jnp.int32),
        interpret=_INTERPRET,
    )(pred_logits, bT_p, siamese_logits, bT_s, lab, tgt_boxes)
    loss16 = _make_sc_loss()(idx, pred_queries.reshape(B * Q, D),
                             siamese_query.reshape(B * Q, D))
    return loss16[0].reshape(())


# speedup vs baseline: 1.0105x; 1.0105x over previous
import functools

import jax
import jax.numpy as jnp
from jax import lax
from jax.experimental import pallas as pl
from jax.experimental.pallas import tpu as pltpu, tpu_sc as plsc

B, Q, C, D, T = 4, 900, 91, 256, 25
P = 2 * B
TPAD = 32
_HIGH = jax.lax.Precision.HIGHEST
_INTERPRET = False


def _cost_T(logits, bT, lab_col, tbox):
    m = jnp.max(logits, axis=1, keepdims=True)
    e = jnp.exp(logits - m)
    prob = e / jnp.sum(e, axis=1, keepdims=True)
    cls_iota = jax.lax.broadcasted_iota(jnp.int32, (T, C), 1)
    onehot = (lab_col == cls_iota).astype(jnp.float32)
    g = jax.lax.dot_general(onehot, prob, (((1,), (1,)), ((), ())),
                            precision=_HIGH)
    cost = -2.0 * g
    for k in range(4):
        cost = cost + 5.0 * jnp.abs(tbox[:, k:k + 1] - bT[k:k + 1, :])
    return cost


def _match_body(lg_p, bT_p, lg_s, bT_s, lab, tb, out_ref):
    costs = []
    for lg, bT in ((lg_p, bT_p), (lg_s, bT_s)):
        for b in range(B):
            costs.append(_cost_T(lg[b], bT[b], lab[b], tb[b]))
    cost3 = jnp.stack(costs, axis=1)

    iota_q = jax.lax.broadcasted_iota(jnp.int32, (P, Q), 1)
    tcol = jax.lax.broadcasted_iota(jnp.int32, (P, T), 1)
    avail = jnp.ones((P, Q), jnp.float32)
    I = jnp.zeros((P, T), jnp.int32)
    for t in range(T):
        col = jnp.where(avail > 0.0, cost3[t], jnp.inf)
        mval = jnp.min(col, axis=1, keepdims=True)
        idx = jnp.min(jnp.where(col == mval, iota_q, jnp.int32(2 ** 30)),
                      axis=1, keepdims=True)
        avail = jnp.where(iota_q == idx, 0.0, avail)
        I = jnp.where(tcol == t, idx, I)

    base = (jax.lax.broadcasted_iota(jnp.int32, (P, T), 0) % B) * Q
    out_ref[:, 0:T] = I + base
    out_ref[:, T:TPAD] = jnp.zeros((P, TPAD - T), jnp.int32)


def _newton_rsqrt(x):
    xi = plsc.bitcast(x, jnp.int32)
    y = plsc.bitcast(jnp.int32(0x5F3759DF) - (xi >> 1), jnp.float32)
    for _ in range(4):
        y = y * (1.5 - 0.5 * x * y * y)
    return y


_SEG = 8


def _sc_loss_body(idx_hbm, qp_hbm, qs_hbm, out_hbm,
                  idx1_v, idx2_v, f1_v, f2_v, red_v, red16_v,
                  csum_sh, sem):
    c = lax.axis_index("c")
    s = lax.axis_index("s")
    lane = lax.iota(jnp.int32, 16)

    @pl.when(c == 0)
    def _pair_and_reduce():
        b = s // 4
        tlo = (s % 4) * _SEG
        pltpu.sync_copy(idx_hbm.at[b, pl.ds(tlo, _SEG)], idx1_v)
        pltpu.sync_copy(idx_hbm.at[b + B, pl.ds(tlo, _SEG)], idx2_v)
        cp1 = pltpu.async_copy(qp_hbm.at[idx1_v], f1_v, sem)
        cp2 = pltpu.async_copy(qs_hbm.at[idx2_v], f2_v, sem)
        cp1.wait()
        cp2.wait()
        adv = jnp.zeros((16,), jnp.float32)
        a1v = jnp.ones((16,), jnp.float32)
        a2v = jnp.ones((16,), jnp.float32)
        for tl in range(_SEG):
            ad = jnp.zeros((16,), jnp.float32)
            a1 = jnp.zeros((16,), jnp.float32)
            a2 = jnp.zeros((16,), jnp.float32)
            for dc in range(D // 16):
                f1 = f1_v[tl, pl.ds(dc * 16, 16)]
                f2 = f2_v[tl, pl.ds(dc * 16, 16)]
                ad = ad + f1 * f2
                a1 = a1 + f1 * f1
                a2 = a2 + f2 * f2
            sel = lane == tl
            adv = jnp.where(sel, jnp.sum(ad), adv)
            a1v = jnp.where(sel, jnp.sum(a1), a1v)
            a2v = jnp.where(sel, jnp.sum(a2), a2v)
        cosv = adv * _newton_rsqrt(a1v * a2v)
        valid = (tlo + lane) < T
        csc = jnp.sum(jnp.where(valid & (lane < _SEG), cosv, 0.0))
        red_v[...] = jnp.where(lane == 0, csc, 0.0)
        pltpu.sync_copy(red_v, csum_sh.at[s])

    plsc.subcore_barrier()

    @pl.when((c == 0) & (s == 0))
    def _finalize():
        pltpu.sync_copy(csum_sh, red16_v)
        tot = jnp.zeros((16,), jnp.float32)
        for w in range(16):
            tot = tot + red16_v[w]
        total = jnp.sum(tot)
        red_v[...] = jnp.where(lane == 0, -total * (1.0 / (B * T)), 0.0)
        pltpu.sync_copy(red_v, out_hbm)


def _make_sc_loss():
    return functools.partial(
        pl.kernel,
        out_type=jax.ShapeDtypeStruct((16,), jnp.float32),
        mesh=plsc.VectorSubcoreMesh(core_axis_name="c", subcore_axis_name="s"),
        compiler_params=pltpu.CompilerParams(use_tc_tiling_on_sc=False,
                                             needs_layout_passes=False),
        interpret=_INTERPRET,
        scratch_types=[
            pltpu.VMEM((_SEG,), jnp.int32),
            pltpu.VMEM((_SEG,), jnp.int32),
            pltpu.VMEM((_SEG, D), jnp.float32),
            pltpu.VMEM((_SEG, D), jnp.float32),
            pltpu.VMEM((16,), jnp.float32),
            pltpu.VMEM((16, 16), jnp.float32),
            pltpu.VMEM_SHARED((16, 16), jnp.float32),
            pltpu.SemaphoreType.DMA,
        ],
    )(_sc_loss_body)


@jax.jit
def kernel(pred_logits, pred_boxes, pred_queries, siamese_logits,
           siamese_boxes, siamese_query, tgt_labels, tgt_boxes):
    bT_p = pred_boxes.transpose(0, 2, 1)
    bT_s = siamese_boxes.transpose(0, 2, 1)
    lab = tgt_labels.astype(jnp.int32).reshape(B, T, 1)
    idx = pl.pallas_call(
        _match_body,
        out_shape=jax.ShapeDtypeStruct((P, TPAD), jnp.int32),
        interpret=_INTERPRET,
    )(pred_logits, bT_p, siamese_logits, bT_s, lab, tgt_boxes)
    loss16 = _make_sc_loss()(idx, pred_queries.reshape(B * Q, D),
                             siamese_query.reshape(B * Q, D))
    return loss16[0].reshape(())


# speedup vs baseline: 1.0127x; 1.0022x over previous
import functools

import jax
import jax.numpy as jnp
from jax import lax
from jax.experimental import pallas as pl
from jax.experimental.pallas import tpu as pltpu, tpu_sc as plsc

B, Q, C, D, T = 4, 900, 91, 256, 25
P = 2 * B
TPAD = 32
_HIGH = jax.lax.Precision.HIGHEST
_INTERPRET = False


def _cost_T(logits, bT, lab_col, tbox):
    m = jnp.max(logits, axis=1, keepdims=True)
    e = jnp.exp(logits - m)
    prob = e / jnp.sum(e, axis=1, keepdims=True)
    cls_iota = jax.lax.broadcasted_iota(jnp.int32, (T, C), 1)
    onehot = (lab_col == cls_iota).astype(jnp.float32)
    g = jax.lax.dot_general(onehot, prob, (((1,), (1,)), ((), ())),
                            precision=_HIGH)
    cost = -2.0 * g
    for k in range(4):
        cost = cost + 5.0 * jnp.abs(tbox[:, k:k + 1] - bT[k:k + 1, :])
    return cost


def _match_body(lg_p, bT_p, lg_s, bT_s, lab, tb, out_ref):
    costs = []
    for lg, bT in ((lg_p, bT_p), (lg_s, bT_s)):
        for b in range(B):
            costs.append(_cost_T(lg[b], bT[b], lab[b], tb[b]))
    cost3 = jnp.stack(costs, axis=1)

    iota_q = jax.lax.broadcasted_iota(jnp.int32, (P, Q), 1)
    tcol = jax.lax.broadcasted_iota(jnp.int32, (P, T), 1)
    avail = jnp.ones((P, Q), jnp.float32)
    I = jnp.zeros((P, T), jnp.int32)
    for t in range(T):
        col = jnp.where(avail > 0.0, cost3[t], jnp.inf)
        mval = jnp.min(col, axis=1, keepdims=True)
        idx = jnp.min(jnp.where(col == mval, iota_q, jnp.int32(2 ** 30)),
                      axis=1, keepdims=True)
        avail = jnp.where(iota_q == idx, 0.0, avail)
        I = jnp.where(tcol == t, idx, I)

    base = (jax.lax.broadcasted_iota(jnp.int32, (P, T), 0) % B) * Q
    out_ref[:, 0:T] = I + base
    out_ref[:, T:TPAD] = jnp.zeros((P, TPAD - T), jnp.int32)


def _newton_rsqrt(x):
    xi = plsc.bitcast(x, jnp.int32)
    y = plsc.bitcast(jnp.int32(0x5F3759DF) - (xi >> 1), jnp.float32)
    for _ in range(4):
        y = y * (1.5 - 0.5 * x * y * y)
    return y


_SEG = 8


def _sc_loss_body(idx_hbm, qp_hbm, qs_hbm, out_hbm,
                  idx1_v, idx2_v, f1_v, f2_v, red_v, red16_v,
                  csum_sh, sem):
    c = lax.axis_index("c")
    s = lax.axis_index("s")
    lane = lax.iota(jnp.int32, 16)

    @pl.when(c == 0)
    def _pair_and_reduce():
        b = s // 4
        tlo = (s % 4) * _SEG
        pltpu.sync_copy(idx_hbm.at[b, pl.ds(tlo, _SEG)], idx1_v)
        pltpu.sync_copy(idx_hbm.at[b + B, pl.ds(tlo, _SEG)], idx2_v)
        cp1 = pltpu.async_copy(qp_hbm.at[idx1_v], f1_v, sem)
        cp2 = pltpu.async_copy(qs_hbm.at[idx2_v], f2_v, sem)
        cp1.wait()
        cp2.wait()
        adv = jnp.zeros((16,), jnp.float32)
        a1v = jnp.ones((16,), jnp.float32)
        a2v = jnp.ones((16,), jnp.float32)
        for tl in range(_SEG):
            ad = jnp.zeros((16,), jnp.float32)
            a1 = jnp.zeros((16,), jnp.float32)
            a2 = jnp.zeros((16,), jnp.float32)
            for dc in range(D // 16):
                f1 = f1_v[tl, pl.ds(dc * 16, 16)]
                f2 = f2_v[tl, pl.ds(dc * 16, 16)]
                ad = ad + f1 * f2
                a1 = a1 + f1 * f1
                a2 = a2 + f2 * f2
            sel = lane == tl
            adv = jnp.where(sel, jnp.sum(ad), adv)
            a1v = jnp.where(sel, jnp.sum(a1), a1v)
            a2v = jnp.where(sel, jnp.sum(a2), a2v)
        cosv = adv * _newton_rsqrt(a1v * a2v)
        valid = (tlo + lane) < T
        csc = jnp.sum(jnp.where(valid & (lane < _SEG), cosv, 0.0))
        red_v[...] = jnp.where(lane == 0, csc, 0.0)
        pltpu.sync_copy(red_v, csum_sh.at[s])

    plsc.subcore_barrier()

    @pl.when((c == 0) & (s == 0))
    def _finalize():
        pltpu.sync_copy(csum_sh, red16_v)
        tot = jnp.zeros((16,), jnp.float32)
        for w in range(16):
            tot = tot + red16_v[w]
        total = jnp.sum(tot)
        red_v[...] = jnp.where(lane == 0, -total * (1.0 / (B * T)), 0.0)
        pltpu.sync_copy(red_v, out_hbm)


def _make_sc_loss():
    return functools.partial(
        pl.kernel,
        out_type=jax.ShapeDtypeStruct((16,), jnp.float32),
        mesh=plsc.VectorSubcoreMesh(core_axis_name="c", subcore_axis_name="s"),
        compiler_params=pltpu.CompilerParams(use_tc_tiling_on_sc=False,
                                             needs_layout_passes=False,
                                             skip_device_barrier=True),
        interpret=_INTERPRET,
        scratch_types=[
            pltpu.VMEM((_SEG,), jnp.int32),
            pltpu.VMEM((_SEG,), jnp.int32),
            pltpu.VMEM((_SEG, D), jnp.float32),
            pltpu.VMEM((_SEG, D), jnp.float32),
            pltpu.VMEM((16,), jnp.float32),
            pltpu.VMEM((16, 16), jnp.float32),
            pltpu.VMEM_SHARED((16, 16), jnp.float32),
            pltpu.SemaphoreType.DMA,
        ],
    )(_sc_loss_body)


@jax.jit
def kernel(pred_logits, pred_boxes, pred_queries, siamese_logits,
           siamese_boxes, siamese_query, tgt_labels, tgt_boxes):
    bT_p = pred_boxes.transpose(0, 2, 1)
    bT_s = siamese_boxes.transpose(0, 2, 1)
    lab = tgt_labels.astype(jnp.int32).reshape(B, T, 1)
    idx = pl.pallas_call(
        _match_body,
        out_shape=jax.ShapeDtypeStruct((P, TPAD), jnp.int32),
        interpret=_INTERPRET,
    )(pred_logits, bT_p, siamese_logits, bT_s, lab, tgt_boxes)
    loss16 = _make_sc_loss()(idx, pred_queries.reshape(B * Q, D),
                             siamese_query.reshape(B * Q, D))
    return loss16[0].reshape(())


# speedup vs baseline: 1.0425x; 1.0294x over previous
import functools

import jax
import jax.numpy as jnp
from jax import lax
from jax.experimental import pallas as pl
from jax.experimental.pallas import tpu as pltpu, tpu_sc as plsc

B, Q, C, D, T = 4, 900, 91, 256, 25
P = 2 * B
TPAD = 32
_HIGH = jax.lax.Precision.HIGHEST
_INTERPRET = False


def _cost_T(logits, bT, lab_col, tbox):
    m = jnp.max(logits, axis=1, keepdims=True)
    e = jnp.exp(logits - m)
    prob = e / jnp.sum(e, axis=1, keepdims=True)
    cls_iota = jax.lax.broadcasted_iota(jnp.int32, (T, C), 1)
    onehot = (lab_col == cls_iota).astype(jnp.float32)
    g = jax.lax.dot_general(onehot, prob, (((1,), (1,)), ((), ())),
                            precision=_HIGH)
    cost = -2.0 * g
    for k in range(4):
        cost = cost + 5.0 * jnp.abs(tbox[:, k:k + 1] - bT[k:k + 1, :])
    return cost


def _match_body(lg_p, bT_p, lg_s, bT_s, lab, tb, out_ref):
    costs = []
    for lg, bT in ((lg_p, bT_p), (lg_s, bT_s)):
        for b in range(B):
            costs.append(_cost_T(lg[b], bT[b], lab[b], tb[b]))
    cost3 = jnp.stack(costs, axis=1)

    iota_q = jax.lax.broadcasted_iota(jnp.int32, (P, Q), 1)
    tcol = jax.lax.broadcasted_iota(jnp.int32, (P, T), 1)
    avail = jnp.ones((P, Q), jnp.float32)
    I = jnp.zeros((P, T), jnp.int32)
    for t in range(T):
        col = jnp.where(avail > 0.0, cost3[t], jnp.inf)
        mval = jnp.min(col, axis=1, keepdims=True)
        idx = jnp.min(jnp.where(col == mval, iota_q, jnp.int32(2 ** 30)),
                      axis=1, keepdims=True)
        avail = jnp.where(iota_q == idx, 0.0, avail)
        I = jnp.where(tcol == t, idx, I)

    base = (jax.lax.broadcasted_iota(jnp.int32, (P, T), 0) % B) * Q
    out_ref[:, 0:T] = I + base
    out_ref[:, T:TPAD] = jnp.zeros((P, TPAD - T), jnp.int32)


def _newton_rsqrt(x):
    xi = plsc.bitcast(x, jnp.int32)
    y = plsc.bitcast(jnp.int32(0x5F3759DF) - (xi >> 1), jnp.float32)
    for _ in range(4):
        y = y * (1.5 - 0.5 * x * y * y)
    return y


_SEG = 8


def _sc_loss_body(idx_hbm, qp_hbm, qs_hbm, out_hbm,
                  idx1_v, idx2_v, f1_v, f2_v, red_v, red16_v,
                  csum_sh, sem):
    c = lax.axis_index("c")
    s = lax.axis_index("s")
    lane = lax.iota(jnp.int32, 16)

    @pl.when(c == 0)
    def _pair_and_reduce():
        b = s // 4
        tlo = (s % 4) * _SEG
        pltpu.sync_copy(idx_hbm.at[b, pl.ds(tlo, _SEG)], idx1_v)
        pltpu.sync_copy(idx_hbm.at[b + B, pl.ds(tlo, _SEG)], idx2_v)
        cp1 = pltpu.async_copy(qp_hbm.at[idx1_v], f1_v, sem)
        cp2 = pltpu.async_copy(qs_hbm.at[idx2_v], f2_v, sem)
        cp1.wait()
        cp2.wait()
        adv = jnp.zeros((16,), jnp.float32)
        a1v = jnp.ones((16,), jnp.float32)
        a2v = jnp.ones((16,), jnp.float32)
        for tl in range(_SEG):
            ad = jnp.zeros((16,), jnp.float32)
            a1 = jnp.zeros((16,), jnp.float32)
            a2 = jnp.zeros((16,), jnp.float32)
            for dc in range(D // 16):
                f1 = f1_v[tl, pl.ds(dc * 16, 16)]
                f2 = f2_v[tl, pl.ds(dc * 16, 16)]
                ad = ad + f1 * f2
                a1 = a1 + f1 * f1
                a2 = a2 + f2 * f2
            sel = lane == tl
            adv = jnp.where(sel, jnp.sum(ad), adv)
            a1v = jnp.where(sel, jnp.sum(a1), a1v)
            a2v = jnp.where(sel, jnp.sum(a2), a2v)
        cosv = adv * _newton_rsqrt(a1v * a2v)
        valid = (tlo + lane) < T
        csc = jnp.sum(jnp.where(valid & (lane < _SEG), cosv, 0.0))
        red_v[...] = jnp.where(lane == 0, csc, 0.0)
        pltpu.sync_copy(red_v, csum_sh.at[s])

    plsc.subcore_barrier()

    @pl.when((c == 0) & (s == 0))
    def _finalize():
        pltpu.sync_copy(csum_sh, red16_v)
        tot = jnp.zeros((16,), jnp.float32)
        for w in range(16):
            tot = tot + red16_v[w]
        total = jnp.sum(tot)
        red_v[...] = jnp.where(lane == 0, -total * (1.0 / (B * T)), 0.0)
        pltpu.sync_copy(red_v, out_hbm)


def _make_sc_loss():
    return functools.partial(
        pl.kernel,
        out_type=jax.ShapeDtypeStruct((16,), jnp.float32),
        mesh=plsc.VectorSubcoreMesh(core_axis_name="c", subcore_axis_name="s",
                                    num_cores=1),
        compiler_params=pltpu.CompilerParams(use_tc_tiling_on_sc=False,
                                             needs_layout_passes=False,
                                             skip_device_barrier=True),
        interpret=_INTERPRET,
        scratch_types=[
            pltpu.VMEM((_SEG,), jnp.int32),
            pltpu.VMEM((_SEG,), jnp.int32),
            pltpu.VMEM((_SEG, D), jnp.float32),
            pltpu.VMEM((_SEG, D), jnp.float32),
            pltpu.VMEM((16,), jnp.float32),
            pltpu.VMEM((16, 16), jnp.float32),
            pltpu.VMEM_SHARED((16, 16), jnp.float32),
            pltpu.SemaphoreType.DMA,
        ],
    )(_sc_loss_body)


@jax.jit
def kernel(pred_logits, pred_boxes, pred_queries, siamese_logits,
           siamese_boxes, siamese_query, tgt_labels, tgt_boxes):
    bT_p = pred_boxes.transpose(0, 2, 1)
    bT_s = siamese_boxes.transpose(0, 2, 1)
    lab = tgt_labels.astype(jnp.int32).reshape(B, T, 1)
    idx = pl.pallas_call(
        _match_body,
        out_shape=jax.ShapeDtypeStruct((P, TPAD), jnp.int32),
        interpret=_INTERPRET,
    )(pred_logits, bT_p, siamese_logits, bT_s, lab, tgt_boxes)
    loss16 = _make_sc_loss()(idx, pred_queries.reshape(B * Q, D),
                             siamese_query.reshape(B * Q, D))
    return loss16[0].reshape(())
